# Initial kernel scaffold; baseline (speedup 1.0000x reference)
import jax
import jax.numpy as jnp
from jax.experimental import pallas as pl


def kernel(partial, coarse, fine1, fine2, gt):
    raise NotImplementedError("write your pallas kernel here")


# speedup vs baseline: 12.2088x; 12.2088x over previous
import jax
import jax.numpy as jnp
from jax.experimental import pallas as pl

_BIG = 1e30
_K1 = 11


def _round_bf16(x):
    return x.astype(jnp.bfloat16).astype(jnp.float32)


def _noisy_sqd_block(a_blk, bt_rounded, b2):
    a2 = None
    ab = None
    for c in range(3):
        ac = a_blk[:, c:c + 1]
        sq = ac * ac
        a2 = sq if c == 0 else a2 + sq
        p = _round_bf16(ac) * bt_rounded[c:c + 1, :]
        ab = p if c == 0 else ab + p
    return jnp.maximum(a2 + b2 - 2.0 * ab, 0.0)


def _exact_sqd_block(a_blk, bt):
    d = None
    for c in range(3):
        diff = a_blk[:, c:c + 1] - bt[c:c + 1, :]
        d = diff * diff if c == 0 else d + diff * diff
    return d


def _sqnorm_cols(bt):
    return (bt[0:1, :] * bt[0:1, :] + bt[1:2, :] * bt[1:2, :]
            + bt[2:3, :] * bt[2:3, :])


def _row_block_size(n):
    return n if n < 256 else 256


def _loss_kernel(coarse_ref, fine1_ref, fine2_ref, partial_ref,
                 gt_t_ref, f2_t_ref, out_ref):
    gt_t = gt_t_ref[0]
    f2_t = f2_t_ref[0]
    m_gt = gt_t.shape[1]
    m_f2 = f2_t.shape[1]
    gt_tr = _round_bf16(gt_t)
    f2_tr = _round_bf16(f2_t)
    b2_gt = _sqnorm_cols(gt_t)
    b2_f2 = _sqnorm_cols(f2_t)

    def chamfer(a_ref, bt_rounded, b2, m):
        n = a_ref.shape[1]
        r = _row_block_size(n)

        def body(i, carry):
            s, colmin = carry
            a_blk = a_ref[0, pl.ds(i * r, r), :]
            d = _noisy_sqd_block(a_blk, bt_rounded, b2)
            s = s + jnp.sum(jnp.min(d, axis=1))
            colmin = jnp.minimum(colmin, jnp.min(d, axis=0))
            return s, colmin

        init = (jnp.float32(0.0), jnp.full((m,), _BIG, jnp.float32))
        s, colmin = jax.lax.fori_loop(0, n // r, body, init)
        return s, jnp.sum(colmin)

    s_c_row, s_c_col = chamfer(coarse_ref, gt_tr, b2_gt, m_gt)
    s_f1_row, s_f1_col = chamfer(fine1_ref, gt_tr, b2_gt, m_gt)
    s_f2_row, s_f2_col = chamfer(fine2_ref, gt_tr, b2_gt, m_gt)

    n_p = partial_ref.shape[1]
    r_p = _row_block_size(n_p)

    def pbody(i, s):
        a_blk = partial_ref[0, pl.ds(i * r_p, r_p), :]
        d = _noisy_sqd_block(a_blk, f2_tr, b2_f2)
        return s + jnp.sum(jnp.sqrt(jnp.min(d, axis=1)))

    s_partial = jax.lax.fori_loop(0, n_p // r_p, pbody, jnp.float32(0.0))

    n_f = fine2_ref.shape[1]
    r_f = _row_block_size(n_f)
    lane_f2 = jax.lax.broadcasted_iota(jnp.int32, (1, m_f2), 1)

    def sbody(i, s):
        a_blk = fine2_ref[0, pl.ds(i * r_f, r_f), :]
        key = _noisy_sqd_block(a_blk, f2_tr, b2_f2)
        val = _exact_sqd_block(a_blk, f2_t)
        acc = jnp.zeros((r_f, 1), jnp.float32)
        for p in range(_K1):
            mn = jnp.min(key, axis=1, keepdims=True)
            tie = jnp.where(key == mn, lane_f2, m_f2)
            jstar = jnp.min(tie, axis=1, keepdims=True)
            onehot = lane_f2 == jstar
            if p > 0:
                picked = jnp.sum(jnp.where(onehot, val, 0.0), axis=1,
                                 keepdims=True)
                acc = acc + picked
            key = jnp.where(onehot, _BIG, key)
        return s + jnp.sum(acc)

    s_smooth = jax.lax.fori_loop(0, n_f // r_f, sbody, jnp.float32(0.0))

    lane = jax.lax.broadcasted_iota(jnp.int32, (1, 128), 1)
    vals = [s_c_row, s_c_col, s_f1_row, s_f1_col,
            s_f2_row, s_f2_col, s_partial, s_smooth]
    out = jnp.zeros((1, 128), jnp.float32)
    for j, v in enumerate(vals):
        out = jnp.where(lane == j, v, out)
    out_ref[...] = out[None]


def kernel(partial, coarse, fine1, fine2, gt):
    b, n_partial, _ = partial.shape
    n_coarse = coarse.shape[1]
    n_fine1 = fine1.shape[1]
    n_fine2 = fine2.shape[1]
    n_gt = gt.shape[1]

    gt_t = jnp.transpose(gt, (0, 2, 1))
    f2_t = jnp.transpose(fine2, (0, 2, 1))

    spec3 = lambda n: pl.BlockSpec((1, n, 3), lambda i: (i, 0, 0))
    spect = lambda n: pl.BlockSpec((1, 3, n), lambda i: (i, 0, 0))

    sums = pl.pallas_call(
        _loss_kernel,
        grid=(b,),
        in_specs=[spec3(n_coarse), spec3(n_fine1), spec3(n_fine2),
                  spec3(n_partial), spect(n_gt), spect(n_fine2)],
        out_specs=pl.BlockSpec((1, 1, 128), lambda i: (i, 0, 0)),
        out_shape=jax.ShapeDtypeStruct((b, 1, 128), jnp.float32),
    )(coarse, fine1, fine2, partial, gt_t, f2_t)
    sums = sums[:, 0, :]

    cd_coarse = jnp.mean(sums[:, 0] / n_coarse + sums[:, 1] / n_gt)
    cd_fine1 = jnp.mean(sums[:, 2] / n_fine1 + sums[:, 3] / n_gt)
    cd_fine2 = jnp.mean(sums[:, 4] / n_fine2 + sums[:, 5] / n_gt)
    partial_loss = jnp.mean(sums[:, 6]) / n_partial
    smooth_loss = jnp.mean(sums[:, 7]) / (n_fine2 * (_K1 - 1))
    total = (cd_coarse + cd_fine1 + cd_fine2 +
             0.5 * partial_loss + 0.1 * smooth_loss)
    return (total, cd_coarse, cd_fine1, cd_fine2, partial_loss, smooth_loss)


# speedup vs baseline: 17.3960x; 1.4249x over previous
import jax
import jax.numpy as jnp
from jax.experimental import pallas as pl

_BIG = 1e30
_K1 = 11


def _sqnorm_rows(a8):
    return (a8[:, 0:1] * a8[:, 0:1] + a8[:, 1:2] * a8[:, 1:2]
            + a8[:, 2:3] * a8[:, 2:3])


def _sqnorm_cols(bt):
    return (bt[0:1, :] * bt[0:1, :] + bt[1:2, :] * bt[1:2, :]
            + bt[2:3, :] * bt[2:3, :])


def _noisy_key_block(a8, bt_bf, b2):
    a2 = _sqnorm_rows(a8)
    ab = jnp.dot(a8.astype(jnp.bfloat16), bt_bf,
                 preferred_element_type=jnp.float32)
    return jnp.maximum(a2 + b2 - 2.0 * ab, 0.0)


def _row_block_size(n):
    return n if n < 256 else 256


def _loss_kernel(coarse_ref, fine1_ref, fine2_ref, partial_ref,
                 gt_t_ref, f2_t_ref, out_ref):
    gt_t = gt_t_ref[0]
    f2_t = f2_t_ref[0]
    gt_bf = gt_t.astype(jnp.bfloat16)
    f2_bf = f2_t.astype(jnp.bfloat16)
    b2_gt = _sqnorm_cols(gt_t)
    b2_f2 = _sqnorm_cols(f2_t)
    m_gt = gt_t.shape[1]

    def chamfer(a_ref, bt_bf, b2, m):
        n = a_ref.shape[1]
        r = _row_block_size(n)

        def body(i, carry):
            s, colmin = carry
            a8 = a_ref[0, pl.ds(i * r, r), :]
            d = _noisy_key_block(a8, bt_bf, b2)
            s = s + jnp.sum(jnp.min(d, axis=1))
            colmin = jnp.minimum(colmin, jnp.min(d, axis=0))
            return s, colmin

        init = (jnp.float32(0.0), jnp.full((m,), _BIG, jnp.float32))
        s, colmin = jax.lax.fori_loop(0, n // r, body, init)
        return s, jnp.sum(colmin)

    s_c_row, s_c_col = chamfer(coarse_ref, gt_bf, b2_gt, m_gt)
    s_f1_row, s_f1_col = chamfer(fine1_ref, gt_bf, b2_gt, m_gt)
    s_f2_row, s_f2_col = chamfer(fine2_ref, gt_bf, b2_gt, m_gt)

    n_p = partial_ref.shape[1]
    r_p = _row_block_size(n_p)

    def pbody(i, s):
        a8 = partial_ref[0, pl.ds(i * r_p, r_p), :]
        d = _noisy_key_block(a8, f2_bf, b2_f2)
        return s + jnp.sum(jnp.sqrt(jnp.min(d, axis=1)))

    s_partial = jax.lax.fori_loop(0, n_p // r_p, pbody, jnp.float32(0.0))

    n_f = fine2_ref.shape[1]
    r_f = _row_block_size(n_f)
    kf = float(_K1)

    def sbody(i, s):
        a8 = fine2_ref[0, pl.ds(i * r_f, r_f), :]
        a2 = _sqnorm_rows(a8)
        ab = jnp.dot(a8.astype(jnp.bfloat16), f2_bf,
                     preferred_element_type=jnp.float32)
        key = jnp.maximum(a2 + b2_f2 - 2.0 * ab, 0.0)
        ab_hi = jax.lax.dot_general(
            a8, f2_t, dimension_numbers=(((1,), (0,)), ((), ())),
            precision=jax.lax.Precision.HIGHEST,
            preferred_element_type=jnp.float32)
        val = jnp.maximum(a2 + b2_f2 - 2.0 * ab_hi, 0.0)

        m1 = jnp.min(key, axis=1, keepdims=True)
        le = key <= m1
        cnt = jnp.sum(jnp.where(le, 1.0, 0.0), axis=1, keepdims=True)
        krem = kf - jnp.minimum(cnt, kf)
        tau = m1
        k = jnp.where(le, _BIG, key)
        for _ in range(_K1 - 1):
            m = jnp.min(k, axis=1, keepdims=True)
            le = k <= m
            cnt = jnp.sum(jnp.where(le, 1.0, 0.0), axis=1, keepdims=True)
            take = jnp.minimum(cnt, krem)
            tau = jnp.where(take > 0.0, m, tau)
            krem = krem - take
            k = jnp.where(le, _BIG, k)

        lt = key < tau
        c_lt = jnp.sum(jnp.where(lt, 1.0, 0.0), axis=1, keepdims=True)
        s_lt = jnp.sum(jnp.where(lt, val, 0.0), axis=1, keepdims=True)
        eq = key == tau
        c_eq = jnp.sum(jnp.where(eq, 1.0, 0.0), axis=1, keepdims=True)
        s_eq = jnp.sum(jnp.where(eq, val, 0.0), axis=1, keepdims=True)
        sel = s_lt + s_eq * ((kf - c_lt) / c_eq)
        eq1 = key == m1
        c1 = jnp.sum(jnp.where(eq1, 1.0, 0.0), axis=1, keepdims=True)
        s1 = jnp.sum(jnp.where(eq1, val, 0.0), axis=1, keepdims=True)
        return s + jnp.sum(sel - s1 / c1)

    s_smooth = jax.lax.fori_loop(0, n_f // r_f, sbody, jnp.float32(0.0))

    lane = jax.lax.broadcasted_iota(jnp.int32, (1, 128), 1)
    vals = [s_c_row, s_c_col, s_f1_row, s_f1_col,
            s_f2_row, s_f2_col, s_partial, s_smooth]
    out = jnp.zeros((1, 128), jnp.float32)
    for j, v in enumerate(vals):
        out = jnp.where(lane == j, v, out)
    out_ref[...] = out[None]


def kernel(partial, coarse, fine1, fine2, gt):
    b, n_partial, _ = partial.shape
    n_coarse = coarse.shape[1]
    n_fine1 = fine1.shape[1]
    n_fine2 = fine2.shape[1]
    n_gt = gt.shape[1]

    def pad_rows(x):
        return jnp.concatenate(
            [x, jnp.zeros((b, x.shape[1], 5), x.dtype)], axis=2)

    def pad_t(x):
        xt = jnp.transpose(x, (0, 2, 1))
        return jnp.concatenate(
            [xt, jnp.zeros((b, 5, x.shape[1]), x.dtype)], axis=1)

    spec3 = lambda n: pl.BlockSpec((1, n, 8), lambda i: (i, 0, 0))
    spect = lambda n: pl.BlockSpec((1, 8, n), lambda i: (i, 0, 0))

    sums = pl.pallas_call(
        _loss_kernel,
        grid=(b,),
        in_specs=[spec3(n_coarse), spec3(n_fine1), spec3(n_fine2),
                  spec3(n_partial), spect(n_gt), spect(n_fine2)],
        out_specs=pl.BlockSpec((1, 1, 128), lambda i: (i, 0, 0)),
        out_shape=jax.ShapeDtypeStruct((b, 1, 128), jnp.float32),
    )(pad_rows(coarse), pad_rows(fine1), pad_rows(fine2), pad_rows(partial),
      pad_t(gt), pad_t(fine2))
    sums = sums[:, 0, :]

    cd_coarse = jnp.mean(sums[:, 0] / n_coarse + sums[:, 1] / n_gt)
    cd_fine1 = jnp.mean(sums[:, 2] / n_fine1 + sums[:, 3] / n_gt)
    cd_fine2 = jnp.mean(sums[:, 4] / n_fine2 + sums[:, 5] / n_gt)
    partial_loss = jnp.mean(sums[:, 6]) / n_partial
    smooth_loss = jnp.mean(sums[:, 7]) / (n_fine2 * (_K1 - 1))
    total = (cd_coarse + cd_fine1 + cd_fine2 +
             0.5 * partial_loss + 0.1 * smooth_loss)
    return (total, cd_coarse, cd_fine1, cd_fine2, partial_loss, smooth_loss)


# speedup vs baseline: 18.6598x; 1.0727x over previous
import jax
import jax.numpy as jnp
from jax.experimental import pallas as pl

_K1 = 11


def _sqnorm_rows(a8):
    return (a8[:, 0:1] * a8[:, 0:1] + a8[:, 1:2] * a8[:, 1:2]
            + a8[:, 2:3] * a8[:, 2:3])


def _sqnorm_cols(bt):
    return (bt[0:1, :] * bt[0:1, :] + bt[1:2, :] * bt[1:2, :]
            + bt[2:3, :] * bt[2:3, :])


def _row_block_size(n):
    return n if n < 256 else 256


def _loss_kernel(coarse_ref, fine1_ref, fine2_ref, partial_ref,
                 gt_t_ref, f2_t_ref, out_ref):
    gt_t = gt_t_ref[0]
    f2_t = f2_t_ref[0]
    gt_bfm2 = (gt_t * -2.0).astype(jnp.bfloat16)
    f2_bfm2 = (f2_t * -2.0).astype(jnp.bfloat16)
    b2_gt = _sqnorm_cols(gt_t)
    b2_f2 = _sqnorm_cols(f2_t)
    m_gt = gt_t.shape[1]
    m_f2 = f2_t.shape[1]

    def chamfer(a_ref, bt_bfm2, b2, m):
        n = a_ref.shape[1]
        r = _row_block_size(n)

        def body(i, carry):
            s, colmin = carry
            a8 = a_ref[0, pl.ds(i * r, r), :]
            a2 = _sqnorm_rows(a8)
            ab2 = jnp.dot(a8.astype(jnp.bfloat16), bt_bfm2,
                          preferred_element_type=jnp.float32)
            e = b2 + ab2
            rmin = jnp.min(e, axis=1, keepdims=True) + a2
            s = s + jnp.sum(jnp.maximum(rmin, 0.0))
            colmin = jnp.minimum(colmin, jnp.min(a2 + e, axis=0))
            return s, colmin

        init = (jnp.float32(0.0), jnp.full((m,), 1e30, jnp.float32))
        s, colmin = jax.lax.fori_loop(0, n // r, body, init)
        return s, jnp.sum(jnp.maximum(colmin, 0.0))

    s_c_row, s_c_col = chamfer(coarse_ref, gt_bfm2, b2_gt, m_gt)
    s_f1_row, s_f1_col = chamfer(fine1_ref, gt_bfm2, b2_gt, m_gt)
    s_f2_row, s_f2_col = chamfer(fine2_ref, gt_bfm2, b2_gt, m_gt)

    n_p = partial_ref.shape[1]
    r_p = _row_block_size(n_p)

    def pbody(i, s):
        a8 = partial_ref[0, pl.ds(i * r_p, r_p), :]
        a2 = _sqnorm_rows(a8)
        ab2 = jnp.dot(a8.astype(jnp.bfloat16), f2_bfm2,
                      preferred_element_type=jnp.float32)
        rmin = jnp.min(b2_f2 + ab2, axis=1, keepdims=True) + a2
        return s + jnp.sum(jnp.sqrt(jnp.maximum(rmin, 0.0)))

    s_partial = jax.lax.fori_loop(0, n_p // r_p, pbody, jnp.float32(0.0))

    b_aug = jnp.concatenate(
        [-2.0 * f2_t, jnp.ones((1, m_f2), jnp.float32), b2_f2,
         jnp.zeros((6, m_f2), jnp.float32)], axis=0)
    n_f = fine2_ref.shape[1]
    r_f = _row_block_size(n_f)
    kf = float(_K1)

    def sbody(i, s):
        a8 = fine2_ref[0, pl.ds(i * r_f, r_f), :]
        a2 = _sqnorm_rows(a8)
        ab2 = jnp.dot(a8.astype(jnp.bfloat16), f2_bfm2,
                      preferred_element_type=jnp.float32)
        keyb = (a2 + (b2_f2 + ab2)).astype(jnp.bfloat16)
        a_aug = jnp.concatenate(
            [a8, a2, jnp.ones((r_f, 1), jnp.float32),
             jnp.zeros((r_f, 6), jnp.float32)], axis=1)
        val_raw = jax.lax.dot_general(
            a_aug, b_aug, dimension_numbers=(((1,), (0,)), ((), ())),
            precision=jax.lax.Precision.HIGHEST,
            preferred_element_type=jnp.float32)
        val = jnp.maximum(val_raw, 0.0)

        one_b = jnp.bfloat16(1.0)
        zero_b = jnp.bfloat16(0.0)
        big_b = jnp.bfloat16(1e30)

        m1 = jnp.min(keyb, axis=1, keepdims=True)
        le = keyb <= m1
        cnt0 = jnp.sum(jnp.where(le, one_b, zero_b), axis=1,
                       keepdims=True).astype(jnp.float32)
        krem = kf - jnp.minimum(cnt0, kf)
        tau = m1
        c_lt = jnp.zeros((r_f, 1), jnp.float32)
        c_eq = cnt0
        k = jnp.where(le, big_b, keyb)
        for _ in range(_K1 - 1):
            m = jnp.min(k, axis=1, keepdims=True)
            le = k <= m
            cnt = jnp.sum(jnp.where(le, one_b, zero_b), axis=1,
                          keepdims=True).astype(jnp.float32)
            take = jnp.minimum(cnt, krem)
            sel_p = take > 0.0
            tau = jnp.where(sel_p, m, tau)
            c_lt = jnp.where(sel_p, kf - krem, c_lt)
            c_eq = jnp.where(sel_p, cnt, c_eq)
            krem = krem - take
            k = jnp.where(le, big_b, k)

        s_lt = jnp.sum(jnp.where(keyb < tau, val, 0.0), axis=1, keepdims=True)
        s_eq = jnp.sum(jnp.where(keyb == tau, val, 0.0), axis=1, keepdims=True)
        s1 = jnp.sum(jnp.where(keyb == m1, val, 0.0), axis=1, keepdims=True)
        sel = s_lt + s_eq * ((kf - c_lt) / c_eq)
        return s + jnp.sum(sel - s1 / cnt0)

    s_smooth = jax.lax.fori_loop(0, n_f // r_f, sbody, jnp.float32(0.0))

    lane = jax.lax.broadcasted_iota(jnp.int32, (1, 128), 1)
    vals = [s_c_row, s_c_col, s_f1_row, s_f1_col,
            s_f2_row, s_f2_col, s_partial, s_smooth]
    out = jnp.zeros((1, 128), jnp.float32)
    for j, v in enumerate(vals):
        out = jnp.where(lane == j, v, out)
    out_ref[...] = out[None]


def kernel(partial, coarse, fine1, fine2, gt):
    b, n_partial, _ = partial.shape
    n_coarse = coarse.shape[1]
    n_fine1 = fine1.shape[1]
    n_fine2 = fine2.shape[1]
    n_gt = gt.shape[1]

    def pad_rows(x):
        return jnp.concatenate(
            [x, jnp.zeros((b, x.shape[1], 5), x.dtype)], axis=2)

    def pad_t(x):
        xt = jnp.transpose(x, (0, 2, 1))
        return jnp.concatenate(
            [xt, jnp.zeros((b, 5, x.shape[1]), x.dtype)], axis=1)

    spec3 = lambda n: pl.BlockSpec((1, n, 8), lambda i: (i, 0, 0))
    spect = lambda n: pl.BlockSpec((1, 8, n), lambda i: (i, 0, 0))

    sums = pl.pallas_call(
        _loss_kernel,
        grid=(b,),
        in_specs=[spec3(n_coarse), spec3(n_fine1), spec3(n_fine2),
                  spec3(n_partial), spect(n_gt), spect(n_fine2)],
        out_specs=pl.BlockSpec((1, 1, 128), lambda i: (i, 0, 0)),
        out_shape=jax.ShapeDtypeStruct((b, 1, 128), jnp.float32),
    )(pad_rows(coarse), pad_rows(fine1), pad_rows(fine2), pad_rows(partial),
      pad_t(gt), pad_t(fine2))
    sums = sums[:, 0, :]

    cd_coarse = jnp.mean(sums[:, 0] / n_coarse + sums[:, 1] / n_gt)
    cd_fine1 = jnp.mean(sums[:, 2] / n_fine1 + sums[:, 3] / n_gt)
    cd_fine2 = jnp.mean(sums[:, 4] / n_fine2 + sums[:, 5] / n_gt)
    partial_loss = jnp.mean(sums[:, 6]) / n_partial
    smooth_loss = jnp.mean(sums[:, 7]) / (n_fine2 * (_K1 - 1))
    total = (cd_coarse + cd_fine1 + cd_fine2 +
             0.5 * partial_loss + 0.1 * smooth_loss)
    return (total, cd_coarse, cd_fine1, cd_fine2, partial_loss, smooth_loss)


# speedup vs baseline: 21.9552x; 1.1766x over previous
import jax
import jax.numpy as jnp
from jax.experimental import pallas as pl

_K1 = 11


def _sqnorm_rows(a8):
    return (a8[:, 0:1] * a8[:, 0:1] + a8[:, 1:2] * a8[:, 1:2]
            + a8[:, 2:3] * a8[:, 2:3])


def _sqnorm_cols(bt):
    return (bt[0:1, :] * bt[0:1, :] + bt[1:2, :] * bt[1:2, :]
            + bt[2:3, :] * bt[2:3, :])


def _row_block_size(n):
    return n if n < 256 else 256


def _loss_kernel(coarse_ref, fine1_ref, fine2_ref, partial_ref,
                 gt_t_ref, f2_t_ref, out_ref):
    gt_t = gt_t_ref[0]
    f2_t = f2_t_ref[0]
    gt_bfm2 = (gt_t * -2.0).astype(jnp.bfloat16)
    f2_bfm2 = (f2_t * -2.0).astype(jnp.bfloat16)
    b2_gt = _sqnorm_cols(gt_t)
    b2_f2 = _sqnorm_cols(f2_t)
    m_gt = gt_t.shape[1]
    m_f2 = f2_t.shape[1]

    def chamfer(a_ref, bt_bfm2, b2, m):
        n = a_ref.shape[1]
        r = _row_block_size(n)

        def body(i, carry):
            s, colmin = carry
            a8 = a_ref[0, pl.ds(i * r, r), :]
            a2 = _sqnorm_rows(a8)
            ab2 = jnp.dot(a8.astype(jnp.bfloat16), bt_bfm2,
                          preferred_element_type=jnp.float32)
            e = b2 + ab2
            rmin = jnp.min(e, axis=1, keepdims=True) + a2
            s = s + jnp.sum(jnp.maximum(rmin, 0.0))
            colmin = jnp.minimum(colmin, jnp.min(a2 + e, axis=0))
            return s, colmin

        init = (jnp.float32(0.0), jnp.full((m,), 1e30, jnp.float32))
        s, colmin = jax.lax.fori_loop(0, n // r, body, init)
        return s, jnp.sum(jnp.maximum(colmin, 0.0))

    s_c_row, s_c_col = chamfer(coarse_ref, gt_bfm2, b2_gt, m_gt)
    s_f1_row, s_f1_col = chamfer(fine1_ref, gt_bfm2, b2_gt, m_gt)
    s_f2_row, s_f2_col = chamfer(fine2_ref, gt_bfm2, b2_gt, m_gt)

    n_p = partial_ref.shape[1]
    r_p = _row_block_size(n_p)

    def pbody(i, s):
        a8 = partial_ref[0, pl.ds(i * r_p, r_p), :]
        a2 = _sqnorm_rows(a8)
        ab2 = jnp.dot(a8.astype(jnp.bfloat16), f2_bfm2,
                      preferred_element_type=jnp.float32)
        rmin = jnp.min(b2_f2 + ab2, axis=1, keepdims=True) + a2
        return s + jnp.sum(jnp.sqrt(jnp.maximum(rmin, 0.0)))

    s_partial = jax.lax.fori_loop(0, n_p // r_p, pbody, jnp.float32(0.0))

    b_aug = jnp.concatenate(
        [-2.0 * f2_t, jnp.ones((1, m_f2), jnp.float32), b2_f2,
         jnp.zeros((6, m_f2), jnp.float32)], axis=0)
    n_f = fine2_ref.shape[1]
    r_f = _row_block_size(n_f)
    kf = float(_K1)

    def sbody(i, s):
        a8 = fine2_ref[0, pl.ds(i * r_f, r_f), :]
        a2 = _sqnorm_rows(a8)
        ab2 = jnp.dot(a8.astype(jnp.bfloat16), f2_bfm2,
                      preferred_element_type=jnp.float32)
        keyb = (a2 + (b2_f2 + ab2)).astype(jnp.bfloat16)
        a_aug = jnp.concatenate(
            [a8, a2, jnp.ones((r_f, 1), jnp.float32),
             jnp.zeros((r_f, 6), jnp.float32)], axis=1)
        val_raw = jax.lax.dot_general(
            a_aug, b_aug, dimension_numbers=(((1,), (0,)), ((), ())),
            precision=jax.lax.Precision.HIGHEST,
            preferred_element_type=jnp.float32)
        val = jnp.maximum(val_raw, 0.0)

        one_b = jnp.bfloat16(1.0)
        zero_b = jnp.bfloat16(0.0)
        big_b = jnp.bfloat16(1e30)

        m1 = jnp.min(keyb, axis=1, keepdims=True)
        le = keyb <= m1
        cnt0 = jnp.sum(jnp.where(le, one_b, zero_b), axis=1, keepdims=True,
                       dtype=jnp.bfloat16).astype(jnp.float32)
        krem = kf - jnp.minimum(cnt0, kf)
        tau = m1
        c_lt = jnp.zeros((r_f, 1), jnp.float32)
        c_eq = cnt0
        k = jnp.where(le, big_b, keyb)
        for _ in range(_K1 - 1):
            m = jnp.min(k, axis=1, keepdims=True)
            le = k <= m
            cnt = jnp.sum(jnp.where(le, one_b, zero_b), axis=1, keepdims=True,
                          dtype=jnp.bfloat16).astype(jnp.float32)
            take = jnp.minimum(cnt, krem)
            sel_p = take > 0.0
            tau = jnp.where(sel_p, m, tau)
            c_lt = jnp.where(sel_p, kf - krem, c_lt)
            c_eq = jnp.where(sel_p, cnt, c_eq)
            krem = krem - take
            k = jnp.where(le, big_b, k)

        s_lt = jnp.sum(jnp.where(keyb < tau, val, 0.0), axis=1, keepdims=True)
        s_eq = jnp.sum(jnp.where(keyb == tau, val, 0.0), axis=1, keepdims=True)
        s1 = jnp.sum(jnp.where(keyb == m1, val, 0.0), axis=1, keepdims=True)
        sel = s_lt + s_eq * ((kf - c_lt) / c_eq)
        return s + jnp.sum(sel - s1 / cnt0)

    s_smooth = jax.lax.fori_loop(0, n_f // r_f, sbody, jnp.float32(0.0))

    lane = jax.lax.broadcasted_iota(jnp.int32, (1, 128), 1)
    vals = [s_c_row, s_c_col, s_f1_row, s_f1_col,
            s_f2_row, s_f2_col, s_partial, s_smooth]
    out = jnp.zeros((1, 128), jnp.float32)
    for j, v in enumerate(vals):
        out = jnp.where(lane == j, v, out)
    out_ref[...] = out[None]


def kernel(partial, coarse, fine1, fine2, gt):
    b, n_partial, _ = partial.shape
    n_coarse = coarse.shape[1]
    n_fine1 = fine1.shape[1]
    n_fine2 = fine2.shape[1]
    n_gt = gt.shape[1]

    def pad_rows(x):
        return jnp.concatenate(
            [x, jnp.zeros((b, x.shape[1], 5), x.dtype)], axis=2)

    def pad_t(x):
        xt = jnp.transpose(x, (0, 2, 1))
        return jnp.concatenate(
            [xt, jnp.zeros((b, 5, x.shape[1]), x.dtype)], axis=1)

    spec3 = lambda n: pl.BlockSpec((1, n, 8), lambda i: (i, 0, 0))
    spect = lambda n: pl.BlockSpec((1, 8, n), lambda i: (i, 0, 0))

    sums = pl.pallas_call(
        _loss_kernel,
        grid=(b,),
        in_specs=[spec3(n_coarse), spec3(n_fine1), spec3(n_fine2),
                  spec3(n_partial), spect(n_gt), spect(n_fine2)],
        out_specs=pl.BlockSpec((1, 1, 128), lambda i: (i, 0, 0)),
        out_shape=jax.ShapeDtypeStruct((b, 1, 128), jnp.float32),
    )(pad_rows(coarse), pad_rows(fine1), pad_rows(fine2), pad_rows(partial),
      pad_t(gt), pad_t(fine2))
    sums = sums[:, 0, :]

    cd_coarse = jnp.mean(sums[:, 0] / n_coarse + sums[:, 1] / n_gt)
    cd_fine1 = jnp.mean(sums[:, 2] / n_fine1 + sums[:, 3] / n_gt)
    cd_fine2 = jnp.mean(sums[:, 4] / n_fine2 + sums[:, 5] / n_gt)
    partial_loss = jnp.mean(sums[:, 6]) / n_partial
    smooth_loss = jnp.mean(sums[:, 7]) / (n_fine2 * (_K1 - 1))
    total = (cd_coarse + cd_fine1 + cd_fine2 +
             0.5 * partial_loss + 0.1 * smooth_loss)
    return (total, cd_coarse, cd_fine1, cd_fine2, partial_loss, smooth_loss)


# speedup vs baseline: 26.0135x; 1.1848x over previous
import jax
import jax.numpy as jnp
from jax.experimental import pallas as pl

_K1 = 11


def _sqnorm_rows(a8):
    return (a8[:, 0:1] * a8[:, 0:1] + a8[:, 1:2] * a8[:, 1:2]
            + a8[:, 2:3] * a8[:, 2:3])


def _sqnorm_cols(bt):
    return (bt[0:1, :] * bt[0:1, :] + bt[1:2, :] * bt[1:2, :]
            + bt[2:3, :] * bt[2:3, :])


def _row_block_size(n):
    return n if n < 256 else 256


def _loss_kernel(coarse_ref, fine1_ref, fine2_ref, partial_ref,
                 gt_t_ref, f2_t_ref, out_ref):
    gt_t = gt_t_ref[0]
    f2_t = f2_t_ref[0]
    gt_bfm2 = (gt_t * -2.0).astype(jnp.bfloat16)
    f2_bfm2 = (f2_t * -2.0).astype(jnp.bfloat16)
    b2_gt = _sqnorm_cols(gt_t)
    b2_f2 = _sqnorm_cols(f2_t)
    m_gt = gt_t.shape[1]
    m_f2 = f2_t.shape[1]

    def chamfer(a_ref, bt_bfm2, b2, m):
        n = a_ref.shape[1]
        r = _row_block_size(n)

        def body(i, carry):
            sv, colmin = carry
            a8 = a_ref[0, pl.ds(i * r, r), :]
            a2 = _sqnorm_rows(a8)
            ab2 = jnp.dot(a8.astype(jnp.bfloat16), bt_bfm2,
                          preferred_element_type=jnp.float32)
            e = b2 + ab2
            rmin = jnp.min(e, axis=1, keepdims=True) + a2
            sv = sv + jnp.maximum(rmin, 0.0)
            colmin = jnp.minimum(colmin, jnp.min(a2 + e, axis=0))
            return sv, colmin

        init = (jnp.zeros((r, 1), jnp.float32),
                jnp.full((m,), 1e30, jnp.float32))
        sv, colmin = jax.lax.fori_loop(0, n // r, body, init)
        return jnp.sum(sv), jnp.sum(jnp.maximum(colmin, 0.0))

    s_c_row, s_c_col = chamfer(coarse_ref, gt_bfm2, b2_gt, m_gt)
    s_f1_row, s_f1_col = chamfer(fine1_ref, gt_bfm2, b2_gt, m_gt)
    s_f2_row, s_f2_col = chamfer(fine2_ref, gt_bfm2, b2_gt, m_gt)

    n_p = partial_ref.shape[1]
    r_p = _row_block_size(n_p)

    def pbody(i, sv):
        a8 = partial_ref[0, pl.ds(i * r_p, r_p), :]
        a2 = _sqnorm_rows(a8)
        ab2 = jnp.dot(a8.astype(jnp.bfloat16), f2_bfm2,
                      preferred_element_type=jnp.float32)
        rmin = jnp.min(b2_f2 + ab2, axis=1, keepdims=True) + a2
        return sv + jnp.sqrt(jnp.maximum(rmin, 0.0))

    s_partial = jnp.sum(jax.lax.fori_loop(
        0, n_p // r_p, pbody, jnp.zeros((r_p, 1), jnp.float32)))

    b_aug = jnp.concatenate(
        [-2.0 * f2_t, jnp.ones((1, m_f2), jnp.float32), b2_f2,
         jnp.zeros((6, m_f2), jnp.float32)], axis=0)
    n_f = fine2_ref.shape[1]
    r_f = _row_block_size(n_f)
    kf = float(_K1)

    def sbody(i, sv):
        a8 = fine2_ref[0, pl.ds(i * r_f, r_f), :]
        a2 = _sqnorm_rows(a8)
        ab2 = jnp.dot(a8.astype(jnp.bfloat16), f2_bfm2,
                      preferred_element_type=jnp.float32)
        keyb = (a2 + (b2_f2 + ab2)).astype(jnp.bfloat16)
        a_aug = jnp.concatenate(
            [a8, a2, jnp.ones((r_f, 1), jnp.float32),
             jnp.zeros((r_f, 6), jnp.float32)], axis=1)
        val_raw = jax.lax.dot_general(
            a_aug, b_aug, dimension_numbers=(((1,), (0,)), ((), ())),
            precision=jax.lax.Precision.HIGHEST,
            preferred_element_type=jnp.float32)
        val = jnp.maximum(val_raw, 0.0)

        one_b = jnp.bfloat16(1.0)
        zero_b = jnp.bfloat16(0.0)
        big_b = jnp.bfloat16(1e30)

        m1 = jnp.min(keyb, axis=1, keepdims=True)
        le = keyb <= m1
        cnt0 = jnp.sum(jnp.where(le, one_b, zero_b), axis=1, keepdims=True,
                       dtype=jnp.bfloat16).astype(jnp.float32)
        krem = kf - jnp.minimum(cnt0, kf)
        tau = m1
        c_lt = jnp.zeros((r_f, 1), jnp.float32)
        c_eq = cnt0
        k = jnp.where(le, big_b, keyb)
        for p in range(_K1 - 1):
            m = jnp.min(k, axis=1, keepdims=True)
            le = k <= m
            cnt = jnp.sum(jnp.where(le, one_b, zero_b), axis=1, keepdims=True,
                          dtype=jnp.bfloat16).astype(jnp.float32)
            take = jnp.minimum(cnt, krem)
            sel_p = take > 0.0
            tau = jnp.where(sel_p, m, tau)
            c_lt = jnp.where(sel_p, kf - krem, c_lt)
            c_eq = jnp.where(sel_p, cnt, c_eq)
            krem = krem - take
            if p < _K1 - 2:
                k = jnp.where(le, big_b, k)

        frac = ((kf - c_lt) / c_eq).astype(jnp.bfloat16)
        beta = (1.0 / cnt0).astype(jnp.bfloat16)
        w = jnp.where(keyb < tau, one_b, zero_b)
        w = w + jnp.where(keyb == tau, frac, zero_b)
        w = w - jnp.where(keyb == m1, beta, zero_b)
        picked = jnp.sum(w.astype(jnp.float32) * val, axis=1, keepdims=True)
        return sv + picked

    s_smooth = jnp.sum(jax.lax.fori_loop(
        0, n_f // r_f, sbody, jnp.zeros((r_f, 1), jnp.float32)))

    lane = jax.lax.broadcasted_iota(jnp.int32, (1, 128), 1)
    vals = [s_c_row, s_c_col, s_f1_row, s_f1_col,
            s_f2_row, s_f2_col, s_partial, s_smooth]
    out = jnp.zeros((1, 128), jnp.float32)
    for j, v in enumerate(vals):
        out = jnp.where(lane == j, v, out)
    out_ref[...] = out[None]


def kernel(partial, coarse, fine1, fine2, gt):
    b, n_partial, _ = partial.shape
    n_coarse = coarse.shape[1]
    n_fine1 = fine1.shape[1]
    n_fine2 = fine2.shape[1]
    n_gt = gt.shape[1]

    def pad_rows(x):
        return jnp.concatenate(
            [x, jnp.zeros((b, x.shape[1], 5), x.dtype)], axis=2)

    def pad_t(x):
        xt = jnp.transpose(x, (0, 2, 1))
        return jnp.concatenate(
            [xt, jnp.zeros((b, 5, x.shape[1]), x.dtype)], axis=1)

    spec3 = lambda n: pl.BlockSpec((1, n, 8), lambda i: (i, 0, 0))
    spect = lambda n: pl.BlockSpec((1, 8, n), lambda i: (i, 0, 0))

    sums = pl.pallas_call(
        _loss_kernel,
        grid=(b,),
        in_specs=[spec3(n_coarse), spec3(n_fine1), spec3(n_fine2),
                  spec3(n_partial), spect(n_gt), spect(n_fine2)],
        out_specs=pl.BlockSpec((1, 1, 128), lambda i: (i, 0, 0)),
        out_shape=jax.ShapeDtypeStruct((b, 1, 128), jnp.float32),
    )(pad_rows(coarse), pad_rows(fine1), pad_rows(fine2), pad_rows(partial),
      pad_t(gt), pad_t(fine2))
    sums = sums[:, 0, :]

    cd_coarse = jnp.mean(sums[:, 0] / n_coarse + sums[:, 1] / n_gt)
    cd_fine1 = jnp.mean(sums[:, 2] / n_fine1 + sums[:, 3] / n_gt)
    cd_fine2 = jnp.mean(sums[:, 4] / n_fine2 + sums[:, 5] / n_gt)
    partial_loss = jnp.mean(sums[:, 6]) / n_partial
    smooth_loss = jnp.mean(sums[:, 7]) / (n_fine2 * (_K1 - 1))
    total = (cd_coarse + cd_fine1 + cd_fine2 +
             0.5 * partial_loss + 0.1 * smooth_loss)
    return (total, cd_coarse, cd_fine1, cd_fine2, partial_loss, smooth_loss)


# speedup vs baseline: 26.8517x; 1.0322x over previous
import jax
import jax.numpy as jnp
from jax.experimental import pallas as pl

_K1 = 11


def _sqnorm_rows(a8):
    return (a8[:, 0:1] * a8[:, 0:1] + a8[:, 1:2] * a8[:, 1:2]
            + a8[:, 2:3] * a8[:, 2:3])


def _sqnorm_cols(bt):
    return (bt[0:1, :] * bt[0:1, :] + bt[1:2, :] * bt[1:2, :]
            + bt[2:3, :] * bt[2:3, :])


def _row_block_size(n):
    return n if n < 512 else 512


def _loss_kernel(coarse_ref, fine1_ref, fine2_ref, partial_ref,
                 gt_t_ref, f2_t_ref, out_ref):
    gt_t = gt_t_ref[0]
    f2_t = f2_t_ref[0]
    gt_bfm2 = (gt_t * -2.0).astype(jnp.bfloat16)
    f2_bfm2 = (f2_t * -2.0).astype(jnp.bfloat16)
    b2_gt = _sqnorm_cols(gt_t)
    b2_f2 = _sqnorm_cols(f2_t)
    m_gt = gt_t.shape[1]
    m_f2 = f2_t.shape[1]

    def chamfer(a_ref, bt_bfm2, b2, m):
        n = a_ref.shape[1]
        r = _row_block_size(n)

        def body(i, carry):
            sv, colmin = carry
            a8 = a_ref[0, pl.ds(i * r, r), :]
            a2 = _sqnorm_rows(a8)
            ab2 = jnp.dot(a8.astype(jnp.bfloat16), bt_bfm2,
                          preferred_element_type=jnp.float32)
            e = b2 + ab2
            rmin = jnp.min(e, axis=1, keepdims=True) + a2
            sv = sv + jnp.maximum(rmin, 0.0)
            colmin = jnp.minimum(colmin, jnp.min(a2 + e, axis=0))
            return sv, colmin

        init = (jnp.zeros((r, 1), jnp.float32),
                jnp.full((m,), 1e30, jnp.float32))
        sv, colmin = jax.lax.fori_loop(0, n // r, body, init)
        return jnp.sum(sv), jnp.sum(jnp.maximum(colmin, 0.0))

    s_c_row, s_c_col = chamfer(coarse_ref, gt_bfm2, b2_gt, m_gt)
    s_f1_row, s_f1_col = chamfer(fine1_ref, gt_bfm2, b2_gt, m_gt)
    s_f2_row, s_f2_col = chamfer(fine2_ref, gt_bfm2, b2_gt, m_gt)

    n_p = partial_ref.shape[1]
    r_p = _row_block_size(n_p)

    def pbody(i, sv):
        a8 = partial_ref[0, pl.ds(i * r_p, r_p), :]
        a2 = _sqnorm_rows(a8)
        ab2 = jnp.dot(a8.astype(jnp.bfloat16), f2_bfm2,
                      preferred_element_type=jnp.float32)
        rmin = jnp.min(b2_f2 + ab2, axis=1, keepdims=True) + a2
        return sv + jnp.sqrt(jnp.maximum(rmin, 0.0))

    s_partial = jnp.sum(jax.lax.fori_loop(
        0, n_p // r_p, pbody, jnp.zeros((r_p, 1), jnp.float32)))

    b_aug = jnp.concatenate(
        [-2.0 * f2_t, jnp.ones((1, m_f2), jnp.float32), b2_f2,
         jnp.zeros((6, m_f2), jnp.float32)], axis=0)
    n_f = fine2_ref.shape[1]
    r_f = _row_block_size(n_f)
    kf = float(_K1)

    def sbody(i, sv):
        a8 = fine2_ref[0, pl.ds(i * r_f, r_f), :]
        a2 = _sqnorm_rows(a8)
        ab2 = jnp.dot(a8.astype(jnp.bfloat16), f2_bfm2,
                      preferred_element_type=jnp.float32)
        keyb = (a2 + (b2_f2 + ab2)).astype(jnp.bfloat16)
        a_aug = jnp.concatenate(
            [a8, a2, jnp.ones((r_f, 1), jnp.float32),
             jnp.zeros((r_f, 6), jnp.float32)], axis=1)
        val_raw = jax.lax.dot_general(
            a_aug, b_aug, dimension_numbers=(((1,), (0,)), ((), ())),
            precision=jax.lax.Precision.HIGHEST,
            preferred_element_type=jnp.float32)
        val = jnp.maximum(val_raw, 0.0)

        one_b = jnp.bfloat16(1.0)
        zero_b = jnp.bfloat16(0.0)
        big_b = jnp.bfloat16(1e30)

        m1 = jnp.min(keyb, axis=1, keepdims=True)
        le = keyb <= m1
        cnt0 = jnp.sum(jnp.where(le, one_b, zero_b), axis=1, keepdims=True,
                       dtype=jnp.bfloat16).astype(jnp.float32)
        krem = kf - jnp.minimum(cnt0, kf)
        tau = m1
        c_lt = jnp.zeros((r_f, 1), jnp.float32)
        c_eq = cnt0
        k = jnp.where(le, big_b, keyb)
        for p in range(_K1 - 1):
            m = jnp.min(k, axis=1, keepdims=True)
            le = k <= m
            cnt = jnp.sum(jnp.where(le, one_b, zero_b), axis=1, keepdims=True,
                          dtype=jnp.bfloat16).astype(jnp.float32)
            take = jnp.minimum(cnt, krem)
            sel_p = take > 0.0
            tau = jnp.where(sel_p, m, tau)
            c_lt = jnp.where(sel_p, kf - krem, c_lt)
            c_eq = jnp.where(sel_p, cnt, c_eq)
            krem = krem - take
            if p < _K1 - 2:
                k = jnp.where(le, big_b, k)

        frac = ((kf - c_lt) / c_eq).astype(jnp.bfloat16)
        beta = (1.0 / cnt0).astype(jnp.bfloat16)
        w = jnp.where(keyb < tau, one_b, zero_b)
        w = w + jnp.where(keyb == tau, frac, zero_b)
        w = w - jnp.where(keyb == m1, beta, zero_b)
        picked = jnp.sum(w.astype(jnp.float32) * val, axis=1, keepdims=True)
        return sv + picked

    s_smooth = jnp.sum(jax.lax.fori_loop(
        0, n_f // r_f, sbody, jnp.zeros((r_f, 1), jnp.float32)))

    lane = jax.lax.broadcasted_iota(jnp.int32, (1, 128), 1)
    vals = [s_c_row, s_c_col, s_f1_row, s_f1_col,
            s_f2_row, s_f2_col, s_partial, s_smooth]
    out = jnp.zeros((1, 128), jnp.float32)
    for j, v in enumerate(vals):
        out = jnp.where(lane == j, v, out)
    out_ref[...] = out[None]


def kernel(partial, coarse, fine1, fine2, gt):
    b, n_partial, _ = partial.shape
    n_coarse = coarse.shape[1]
    n_fine1 = fine1.shape[1]
    n_fine2 = fine2.shape[1]
    n_gt = gt.shape[1]

    def pad_rows(x):
        return jnp.concatenate(
            [x, jnp.zeros((b, x.shape[1], 5), x.dtype)], axis=2)

    def pad_t(x):
        xt = jnp.transpose(x, (0, 2, 1))
        return jnp.concatenate(
            [xt, jnp.zeros((b, 5, x.shape[1]), x.dtype)], axis=1)

    spec3 = lambda n: pl.BlockSpec((1, n, 8), lambda i: (i, 0, 0))
    spect = lambda n: pl.BlockSpec((1, 8, n), lambda i: (i, 0, 0))

    sums = pl.pallas_call(
        _loss_kernel,
        grid=(b,),
        in_specs=[spec3(n_coarse), spec3(n_fine1), spec3(n_fine2),
                  spec3(n_partial), spect(n_gt), spect(n_fine2)],
        out_specs=pl.BlockSpec((1, 1, 128), lambda i: (i, 0, 0)),
        out_shape=jax.ShapeDtypeStruct((b, 1, 128), jnp.float32),
    )(pad_rows(coarse), pad_rows(fine1), pad_rows(fine2), pad_rows(partial),
      pad_t(gt), pad_t(fine2))
    sums = sums[:, 0, :]

    cd_coarse = jnp.mean(sums[:, 0] / n_coarse + sums[:, 1] / n_gt)
    cd_fine1 = jnp.mean(sums[:, 2] / n_fine1 + sums[:, 3] / n_gt)
    cd_fine2 = jnp.mean(sums[:, 4] / n_fine2 + sums[:, 5] / n_gt)
    partial_loss = jnp.mean(sums[:, 6]) / n_partial
    smooth_loss = jnp.mean(sums[:, 7]) / (n_fine2 * (_K1 - 1))
    total = (cd_coarse + cd_fine1 + cd_fine2 +
             0.5 * partial_loss + 0.1 * smooth_loss)
    return (total, cd_coarse, cd_fine1, cd_fine2, partial_loss, smooth_loss)


# speedup vs baseline: 30.9682x; 1.1533x over previous
import jax
import jax.numpy as jnp
from jax.experimental import pallas as pl

_K1 = 11


def _sqnorm_rows(a8):
    return (a8[:, 0:1] * a8[:, 0:1] + a8[:, 1:2] * a8[:, 1:2]
            + a8[:, 2:3] * a8[:, 2:3])


def _sqnorm_cols(bt):
    return (bt[0:1, :] * bt[0:1, :] + bt[1:2, :] * bt[1:2, :]
            + bt[2:3, :] * bt[2:3, :])


def _row_block_size(n):
    return n if n < 512 else 512


def _loss_kernel(coarse_ref, fine1_ref, fine2_ref, partial_ref,
                 gt_t_ref, f2_t_ref, out_ref):
    gt_t = gt_t_ref[0]
    f2_t = f2_t_ref[0]
    gt_bfm2 = (gt_t * -2.0).astype(jnp.bfloat16)
    f2_bfm2 = (f2_t * -2.0).astype(jnp.bfloat16)
    b2_gt = _sqnorm_cols(gt_t)
    b2_f2 = _sqnorm_cols(f2_t)
    m_gt = gt_t.shape[1]
    m_f2 = f2_t.shape[1]

    def chamfer(a_ref, bt_bfm2, b2, m):
        n = a_ref.shape[1]
        r = _row_block_size(n)

        def body(i, carry):
            sv, colmin = carry
            a8 = a_ref[0, pl.ds(i * r, r), :]
            a2 = _sqnorm_rows(a8)
            ab2 = jnp.dot(a8.astype(jnp.bfloat16), bt_bfm2,
                          preferred_element_type=jnp.float32)
            e = b2 + ab2
            rmin = jnp.min(e, axis=1, keepdims=True) + a2
            sv = sv + jnp.maximum(rmin, 0.0)
            colmin = jnp.minimum(colmin, jnp.min(a2 + e, axis=0))
            return sv, colmin

        init = (jnp.zeros((r, 1), jnp.float32),
                jnp.full((m,), 1e30, jnp.float32))
        sv, colmin = jax.lax.fori_loop(0, n // r, body, init)
        return jnp.sum(sv), jnp.sum(jnp.maximum(colmin, 0.0))

    s_c_row, s_c_col = chamfer(coarse_ref, gt_bfm2, b2_gt, m_gt)
    s_f1_row, s_f1_col = chamfer(fine1_ref, gt_bfm2, b2_gt, m_gt)
    s_f2_row, s_f2_col = chamfer(fine2_ref, gt_bfm2, b2_gt, m_gt)

    n_p = partial_ref.shape[1]
    r_p = _row_block_size(n_p)

    def pbody(i, sv):
        a8 = partial_ref[0, pl.ds(i * r_p, r_p), :]
        a2 = _sqnorm_rows(a8)
        ab2 = jnp.dot(a8.astype(jnp.bfloat16), f2_bfm2,
                      preferred_element_type=jnp.float32)
        rmin = jnp.min(b2_f2 + ab2, axis=1, keepdims=True) + a2
        return sv + jnp.sqrt(jnp.maximum(rmin, 0.0))

    s_partial = jnp.sum(jax.lax.fori_loop(
        0, n_p // r_p, pbody, jnp.zeros((r_p, 1), jnp.float32)))

    b_aug = jnp.concatenate(
        [-2.0 * f2_t, jnp.ones((1, m_f2), jnp.float32), b2_f2,
         jnp.zeros((6, m_f2), jnp.float32)], axis=0)
    b_hi = b_aug.astype(jnp.bfloat16)
    b_lo = (b_aug - b_hi.astype(jnp.float32)).astype(jnp.bfloat16)
    n_f = fine2_ref.shape[1]
    r_f = _row_block_size(n_f)
    kf = float(_K1)

    def sbody(i, sv):
        a8 = fine2_ref[0, pl.ds(i * r_f, r_f), :]
        a2 = _sqnorm_rows(a8)
        ab2 = jnp.dot(a8.astype(jnp.bfloat16), f2_bfm2,
                      preferred_element_type=jnp.float32)
        keyb = (a2 + (b2_f2 + ab2)).astype(jnp.bfloat16)
        a_aug = jnp.concatenate(
            [a8, a2, jnp.ones((r_f, 1), jnp.float32),
             jnp.zeros((r_f, 6), jnp.float32)], axis=1)
        a_hi = a_aug.astype(jnp.bfloat16)
        a_lo = (a_aug - a_hi.astype(jnp.float32)).astype(jnp.bfloat16)
        val_raw = (jnp.dot(a_hi, b_hi, preferred_element_type=jnp.float32)
                   + (jnp.dot(a_hi, b_lo, preferred_element_type=jnp.float32)
                      + jnp.dot(a_lo, b_hi,
                                preferred_element_type=jnp.float32)))
        val = jnp.maximum(val_raw, 0.0)

        one_b = jnp.bfloat16(1.0)
        zero_b = jnp.bfloat16(0.0)
        big_b = jnp.bfloat16(1e30)

        m1 = jnp.min(keyb, axis=1, keepdims=True)
        le = keyb <= m1
        cnt0 = jnp.sum(jnp.where(le, one_b, zero_b), axis=1, keepdims=True,
                       dtype=jnp.bfloat16).astype(jnp.float32)
        krem = kf - jnp.minimum(cnt0, kf)
        tau = m1
        c_lt = jnp.zeros((r_f, 1), jnp.float32)
        c_eq = cnt0
        k = jnp.where(le, big_b, keyb)
        for p in range(_K1 - 1):
            m = jnp.min(k, axis=1, keepdims=True)
            le = k <= m
            cnt = jnp.sum(jnp.where(le, one_b, zero_b), axis=1, keepdims=True,
                          dtype=jnp.bfloat16).astype(jnp.float32)
            take = jnp.minimum(cnt, krem)
            sel_p = take > 0.0
            tau = jnp.where(sel_p, m, tau)
            c_lt = jnp.where(sel_p, kf - krem, c_lt)
            c_eq = jnp.where(sel_p, cnt, c_eq)
            krem = krem - take
            if p < _K1 - 2:
                k = jnp.where(le, big_b, k)

        frac = ((kf - c_lt) / c_eq).astype(jnp.bfloat16)
        beta = (1.0 / cnt0).astype(jnp.bfloat16)
        w = jnp.where(keyb < tau, one_b, zero_b)
        w = w + jnp.where(keyb == tau, frac, zero_b)
        w = w - jnp.where(keyb == m1, beta, zero_b)
        picked = jnp.sum(w.astype(jnp.float32) * val, axis=1, keepdims=True)
        return sv + picked

    s_smooth = jnp.sum(jax.lax.fori_loop(
        0, n_f // r_f, sbody, jnp.zeros((r_f, 1), jnp.float32)))

    lane = jax.lax.broadcasted_iota(jnp.int32, (1, 128), 1)
    vals = [s_c_row, s_c_col, s_f1_row, s_f1_col,
            s_f2_row, s_f2_col, s_partial, s_smooth]
    out = jnp.zeros((1, 128), jnp.float32)
    for j, v in enumerate(vals):
        out = jnp.where(lane == j, v, out)
    out_ref[...] = out[None]


def kernel(partial, coarse, fine1, fine2, gt):
    b, n_partial, _ = partial.shape
    n_coarse = coarse.shape[1]
    n_fine1 = fine1.shape[1]
    n_fine2 = fine2.shape[1]
    n_gt = gt.shape[1]

    def pad_rows(x):
        return jnp.concatenate(
            [x, jnp.zeros((b, x.shape[1], 5), x.dtype)], axis=2)

    def pad_t(x):
        xt = jnp.transpose(x, (0, 2, 1))
        return jnp.concatenate(
            [xt, jnp.zeros((b, 5, x.shape[1]), x.dtype)], axis=1)

    spec3 = lambda n: pl.BlockSpec((1, n, 8), lambda i: (i, 0, 0))
    spect = lambda n: pl.BlockSpec((1, 8, n), lambda i: (i, 0, 0))

    sums = pl.pallas_call(
        _loss_kernel,
        grid=(b,),
        in_specs=[spec3(n_coarse), spec3(n_fine1), spec3(n_fine2),
                  spec3(n_partial), spect(n_gt), spect(n_fine2)],
        out_specs=pl.BlockSpec((1, 1, 128), lambda i: (i, 0, 0)),
        out_shape=jax.ShapeDtypeStruct((b, 1, 128), jnp.float32),
    )(pad_rows(coarse), pad_rows(fine1), pad_rows(fine2), pad_rows(partial),
      pad_t(gt), pad_t(fine2))
    sums = sums[:, 0, :]

    cd_coarse = jnp.mean(sums[:, 0] / n_coarse + sums[:, 1] / n_gt)
    cd_fine1 = jnp.mean(sums[:, 2] / n_fine1 + sums[:, 3] / n_gt)
    cd_fine2 = jnp.mean(sums[:, 4] / n_fine2 + sums[:, 5] / n_gt)
    partial_loss = jnp.mean(sums[:, 6]) / n_partial
    smooth_loss = jnp.mean(sums[:, 7]) / (n_fine2 * (_K1 - 1))
    total = (cd_coarse + cd_fine1 + cd_fine2 +
             0.5 * partial_loss + 0.1 * smooth_loss)
    return (total, cd_coarse, cd_fine1, cd_fine2, partial_loss, smooth_loss)


# speedup vs baseline: 31.4675x; 1.0161x over previous
import jax
import jax.numpy as jnp
from jax.experimental import pallas as pl

_K1 = 11


def _sqnorm_rows(a8):
    return (a8[:, 0:1] * a8[:, 0:1] + a8[:, 1:2] * a8[:, 1:2]
            + a8[:, 2:3] * a8[:, 2:3])


def _sqnorm_cols(bt):
    return (bt[0:1, :] * bt[0:1, :] + bt[1:2, :] * bt[1:2, :]
            + bt[2:3, :] * bt[2:3, :])


def _row_block_size(n):
    return n if n < 512 else 512


def _loss_kernel(coarse_ref, fine1_ref, fine2_ref, partial_ref,
                 gt_t_ref, f2_t_ref, out_ref):
    gt_t = gt_t_ref[0]
    f2_t = f2_t_ref[0]
    gt_bfm2 = (gt_t * -2.0).astype(jnp.bfloat16)
    f2_bfm2 = (f2_t * -2.0).astype(jnp.bfloat16)
    b2_gt = _sqnorm_cols(gt_t)
    b2_f2 = _sqnorm_cols(f2_t)
    m_gt = gt_t.shape[1]
    m_f2 = f2_t.shape[1]

    def chamfer(a_ref, bt_bfm2, b2, m):
        n = a_ref.shape[1]
        r = _row_block_size(n)

        def body(i, carry):
            sv, colmin = carry
            a8 = a_ref[0, pl.ds(i * r, r), :]
            a2 = _sqnorm_rows(a8)
            ab2 = jnp.dot(a8.astype(jnp.bfloat16), bt_bfm2,
                          preferred_element_type=jnp.float32)
            e = b2 + ab2
            rmin = jnp.min(e, axis=1, keepdims=True) + a2
            sv = sv + jnp.maximum(rmin, 0.0)
            colmin = jnp.minimum(colmin, jnp.min(a2 + e, axis=0))
            return sv, colmin

        init = (jnp.zeros((r, 1), jnp.float32),
                jnp.full((m,), 1e30, jnp.float32))
        sv, colmin = jax.lax.fori_loop(0, n // r, body, init)
        return jnp.sum(sv), jnp.sum(jnp.maximum(colmin, 0.0))

    s_c_row, s_c_col = chamfer(coarse_ref, gt_bfm2, b2_gt, m_gt)
    s_f1_row, s_f1_col = chamfer(fine1_ref, gt_bfm2, b2_gt, m_gt)
    s_f2_row, s_f2_col = chamfer(fine2_ref, gt_bfm2, b2_gt, m_gt)

    n_p = partial_ref.shape[1]
    r_p = _row_block_size(n_p)

    def pbody(i, sv):
        a8 = partial_ref[0, pl.ds(i * r_p, r_p), :]
        a2 = _sqnorm_rows(a8)
        ab2 = jnp.dot(a8.astype(jnp.bfloat16), f2_bfm2,
                      preferred_element_type=jnp.float32)
        rmin = jnp.min(b2_f2 + ab2, axis=1, keepdims=True) + a2
        return sv + jnp.sqrt(jnp.maximum(rmin, 0.0))

    s_partial = jnp.sum(jax.lax.fori_loop(
        0, n_p // r_p, pbody, jnp.zeros((r_p, 1), jnp.float32)))

    b_aug = jnp.concatenate(
        [-2.0 * f2_t, jnp.ones((1, m_f2), jnp.float32), b2_f2,
         jnp.zeros((6, m_f2), jnp.float32)], axis=0)
    b_hi = b_aug.astype(jnp.bfloat16)
    b_lo = (b_aug - b_hi.astype(jnp.float32)).astype(jnp.bfloat16)
    n_f = fine2_ref.shape[1]
    r_f = _row_block_size(n_f)
    kf = float(_K1)

    def sbody(i, sv):
        a8 = fine2_ref[0, pl.ds(i * r_f, r_f), :]
        a2 = _sqnorm_rows(a8)
        ab2 = jnp.dot(a8.astype(jnp.bfloat16), f2_bfm2,
                      preferred_element_type=jnp.float32)
        keyb = (a2 + (b2_f2 + ab2)).astype(jnp.bfloat16)
        a_aug = jnp.concatenate(
            [a8, a2, jnp.ones((r_f, 1), jnp.float32),
             jnp.zeros((r_f, 6), jnp.float32)], axis=1)
        a_hi = a_aug.astype(jnp.bfloat16)
        a_lo = (a_aug - a_hi.astype(jnp.float32)).astype(jnp.bfloat16)
        val_raw = (jnp.dot(a_hi, b_hi, preferred_element_type=jnp.float32)
                   + (jnp.dot(a_hi, b_lo, preferred_element_type=jnp.float32)
                      + jnp.dot(a_lo, b_hi,
                                preferred_element_type=jnp.float32)))
        val = val_raw

        one_b = jnp.bfloat16(1.0)
        zero_b = jnp.bfloat16(0.0)
        big_b = jnp.bfloat16(1e30)

        m1 = jnp.min(keyb, axis=1, keepdims=True)
        le = keyb <= m1
        cnt0 = jnp.sum(jnp.where(le, one_b, zero_b), axis=1, keepdims=True,
                       dtype=jnp.bfloat16).astype(jnp.float32)
        krem = kf - jnp.minimum(cnt0, kf)
        tau = m1
        c_lt = jnp.zeros((r_f, 1), jnp.float32)
        c_eq = cnt0
        k = jnp.where(le, big_b, keyb)
        for p in range(_K1 - 1):
            m = jnp.min(k, axis=1, keepdims=True)
            le = k <= m
            cnt = jnp.sum(jnp.where(le, one_b, zero_b), axis=1, keepdims=True,
                          dtype=jnp.bfloat16).astype(jnp.float32)
            take = jnp.minimum(cnt, krem)
            sel_p = take > 0.0
            tau = jnp.where(sel_p, m, tau)
            c_lt = jnp.where(sel_p, kf - krem, c_lt)
            c_eq = jnp.where(sel_p, cnt, c_eq)
            krem = krem - take
            if p < _K1 - 2:
                k = jnp.where(le, big_b, k)

        frac = ((kf - c_lt) / c_eq).astype(jnp.bfloat16)
        beta = (1.0 / cnt0).astype(jnp.bfloat16)
        w = jnp.where(keyb < tau, one_b, zero_b)
        w = w + jnp.where(keyb == tau, frac, zero_b)
        w = w - jnp.where(keyb == m1, beta, zero_b)
        picked = jnp.sum(w.astype(jnp.float32) * val, axis=1, keepdims=True)
        return sv + picked

    s_smooth = jnp.sum(jax.lax.fori_loop(
        0, n_f // r_f, sbody, jnp.zeros((r_f, 1), jnp.float32)))

    lane = jax.lax.broadcasted_iota(jnp.int32, (1, 128), 1)
    vals = [s_c_row, s_c_col, s_f1_row, s_f1_col,
            s_f2_row, s_f2_col, s_partial, s_smooth]
    out = jnp.zeros((1, 128), jnp.float32)
    for j, v in enumerate(vals):
        out = jnp.where(lane == j, v, out)
    out_ref[...] = out[None]


def kernel(partial, coarse, fine1, fine2, gt):
    b, n_partial, _ = partial.shape
    n_coarse = coarse.shape[1]
    n_fine1 = fine1.shape[1]
    n_fine2 = fine2.shape[1]
    n_gt = gt.shape[1]

    def pad_rows(x):
        return jnp.concatenate(
            [x, jnp.zeros((b, x.shape[1], 5), x.dtype)], axis=2)

    def pad_t(x):
        xt = jnp.transpose(x, (0, 2, 1))
        return jnp.concatenate(
            [xt, jnp.zeros((b, 5, x.shape[1]), x.dtype)], axis=1)

    spec3 = lambda n: pl.BlockSpec((1, n, 8), lambda i: (i, 0, 0))
    spect = lambda n: pl.BlockSpec((1, 8, n), lambda i: (i, 0, 0))

    sums = pl.pallas_call(
        _loss_kernel,
        grid=(b,),
        in_specs=[spec3(n_coarse), spec3(n_fine1), spec3(n_fine2),
                  spec3(n_partial), spect(n_gt), spect(n_fine2)],
        out_specs=pl.BlockSpec((1, 1, 128), lambda i: (i, 0, 0)),
        out_shape=jax.ShapeDtypeStruct((b, 1, 128), jnp.float32),
    )(pad_rows(coarse), pad_rows(fine1), pad_rows(fine2), pad_rows(partial),
      pad_t(gt), pad_t(fine2))
    sums = sums[:, 0, :]

    cd_coarse = jnp.mean(sums[:, 0] / n_coarse + sums[:, 1] / n_gt)
    cd_fine1 = jnp.mean(sums[:, 2] / n_fine1 + sums[:, 3] / n_gt)
    cd_fine2 = jnp.mean(sums[:, 4] / n_fine2 + sums[:, 5] / n_gt)
    partial_loss = jnp.mean(sums[:, 6]) / n_partial
    smooth_loss = jnp.mean(sums[:, 7]) / (n_fine2 * (_K1 - 1))
    total = (cd_coarse + cd_fine1 + cd_fine2 +
             0.5 * partial_loss + 0.1 * smooth_loss)
    return (total, cd_coarse, cd_fine1, cd_fine2, partial_loss, smooth_loss)
